# baseline (speedup 1.0000x reference)
import functools

import jax
import jax.numpy as jnp
from jax import lax
from jax.experimental import pallas as pl
from jax.experimental.pallas import tpu as pltpu
from jax.experimental.pallas import tpu_sc as plsc

D = 32
V = 1000000
B = 4096
T = 200
NW = 32
LANES = 128
NBLK = V // LANES
HALF = V - NBLK * LANES
NK_FULL = NBLK // NW
NK_REM = NBLK % NW

_mesh = plsc.VectorSubcoreMesh(core_axis_name="c", subcore_axis_name="s")


@functools.partial(
    pl.kernel,
    mesh=_mesh,
    compiler_params=pltpu.CompilerParams(needs_layout_passes=False),
    out_type=jax.ShapeDtypeStruct((V // 4, LANES), jnp.float32),
    scratch_types=[
        pltpu.VMEM((2, D, LANES), jnp.float32),
        pltpu.VMEM((2, D, LANES), jnp.float32),
        pltpu.SemaphoreType.DMA((2,)),
        pltpu.SemaphoreType.DMA((2,)),
    ],
)
def _linearize(tab_t, tail4, out_hbm, blk, pck, sem_i, sem_o):
    wid = lax.axis_index("s") * 2 + lax.axis_index("c")
    iota = lax.iota(jnp.int32, 16)
    nk = jnp.where(wid < NK_REM, NK_FULL + 1, NK_FULL)

    def in_desc(k, bb):
        c = wid + NW * k
        return pltpu.make_async_copy(
            tab_t.at[:, pl.ds(c * LANES, LANES)], blk.at[bb], sem_i.at[bb])

    def out_desc(k, bb):
        c = wid + NW * k
        return pltpu.make_async_copy(
            pck.at[bb], out_hbm.at[pl.ds(c * D, D)], sem_o.at[bb])

    def transpose_block(bb, width):
        for g in range(width * D // 16):
            j0, d0 = divmod(16 * g, D)
            p0, mm0 = divmod(16 * g, LANES)
            vals = plsc.load_gather(
                blk.at[bb],
                [d0 + iota, lax.broadcast(jnp.int32(j0), (16,))])
            pck[bb, p0, pl.ds(mm0, 16)] = vals

    in_desc(0, 0).start()

    @pl.loop(0, NK_FULL + 2, step=2)
    def _blocks(ko):
        for bb in range(2):
            k = ko + bb

            @pl.when(k < nk)
            def _step():
                @pl.when(k + 1 < nk)
                def _prefetch():
                    @pl.when(k >= 1)
                    def _():
                        out_desc(k - 1, 1 - bb).wait()
                    in_desc(k + 1, 1 - bb).start()

                in_desc(k, bb).wait()
                transpose_block(bb, LANES)
                out_desc(k, bb).start()

    for bb in range(2):
        @pl.when(lax.rem(nk, 2) == bb)
        def _drain():
            out_desc(nk - 2, bb).wait()
            out_desc(nk - 1, 1 - bb).wait()

    @pl.when(wid == NK_REM)
    def _tail():
        pltpu.sync_copy(tail4, blk.at[0, pl.ds(0, 16), :])
        pltpu.sync_copy(blk.at[0, pl.ds(0, 16), :],
                        out_hbm.at[pl.ds(NBLK * D, HALF * D // LANES)])


@functools.partial(
    pl.kernel,
    mesh=_mesh,
    compiler_params=pltpu.CompilerParams(
        use_tc_tiling_on_sc=False, needs_layout_passes=False),
    out_type=jax.ShapeDtypeStruct((T, 4, NW, 8 * LANES), jnp.float32),
    scratch_types=[
        pltpu.VMEM((LANES * T,), jnp.int32),
        pltpu.VMEM((2, LANES), jnp.int32),
        pltpu.VMEM((2, LANES, D), jnp.float32),
        pltpu.VMEM((2, 4, 8 * LANES), jnp.float32),
        pltpu.SemaphoreType.DMA((2,)),
        pltpu.SemaphoreType.DMA((2,)),
    ],
)
def _gather(idx_hbm, tab_hbm, out_hbm, idx_v, idxc, rows, otile, sem_g, sem_o):
    wid = lax.axis_index("s") * 2 + lax.axis_index("c")
    iota = lax.iota(jnp.int32, 16)
    pltpu.sync_copy(idx_hbm.at[pl.ds(wid * (LANES * T), LANES * T)], idx_v)

    def build_idxc(t, bb):
        for g in range(LANES // 16):
            base = T * 16 * g + T * iota
            idxc[bb, pl.ds(16 * g, 16)] = plsc.load_gather(idx_v, [base + t])

    def gather_desc(bb):
        return pltpu.make_async_copy(
            tab_hbm.at[idxc.at[bb]], rows.at[bb], sem_g.at[bb])

    def out_desc(t, bb):
        return pltpu.make_async_copy(
            otile.at[bb], out_hbm.at[t, :, wid], sem_o.at[bb])

    def transpose_rows(bb):
        for d in range(D):
            s, r = divmod(d, 8)
            for g in range(LANES // 16):
                vals = plsc.load_gather(
                    rows.at[bb],
                    [16 * g + iota, lax.broadcast(jnp.int32(d), (16,))])
                otile[bb, s, pl.ds(r * LANES + 16 * g, 16)] = vals

    build_idxc(0, 0)
    gather_desc(0).start()

    @pl.loop(0, T, step=2)
    def _steps(to):
        for bb in range(2):
            t = to + bb

            @pl.when(t + 1 < T)
            def _prefetch():
                build_idxc(t + 1, 1 - bb)
                @pl.when(t >= 1)
                def _():
                    out_desc(t - 1, 1 - bb).wait()
                gather_desc(1 - bb).start()

            gather_desc(bb).wait()
            transpose_rows(bb)
            out_desc(t, bb).start()

    out_desc(T - 2, 0).wait()
    out_desc(T - 1, 1).wait()


def kernel(x, table):
    idx = x.reshape(-1)
    tail4 = table[NBLK * LANES:].reshape(HALF * D // LANES, LANES)
    tab_lin = _linearize(table.T, tail4).reshape(V, D)
    out4 = _gather(idx, tab_lin)
    out5 = out4.reshape(T, 4, NW, 8, LANES)
    return out5.transpose(2, 4, 0, 1, 3).reshape(B, T, D)


# speedup vs baseline: 1.5146x; 1.5146x over previous
import functools

import jax
import jax.numpy as jnp
from jax import lax
from jax.experimental import pallas as pl
from jax.experimental.pallas import tpu as pltpu
from jax.experimental.pallas import tpu_sc as plsc

D = 32
V = 1000000
B = 4096
T = 200
NW = 32
LANES = 128
NBLK = V // LANES
HALF = V - NBLK * LANES
NK_FULL = NBLK // NW
NK_REM = NBLK % NW

_mesh = plsc.VectorSubcoreMesh(core_axis_name="c", subcore_axis_name="s")


@functools.partial(
    pl.kernel,
    mesh=_mesh,
    compiler_params=pltpu.CompilerParams(needs_layout_passes=False),
    out_type=jax.ShapeDtypeStruct((V // 4, LANES), jnp.float32),
    scratch_types=[
        pltpu.VMEM((2, D, LANES), jnp.float32),
        pltpu.VMEM((2, D, LANES), jnp.float32),
        pltpu.SemaphoreType.DMA((2,)),
        pltpu.SemaphoreType.DMA((2,)),
    ],
)
def _linearize(tab_t, tail4, out_hbm, blk, pck, sem_i, sem_o):
    wid = lax.axis_index("s") * 2 + lax.axis_index("c")
    iota = lax.iota(jnp.int32, 16)
    nk = jnp.where(wid < NK_REM, NK_FULL + 1, NK_FULL)

    def in_desc(k, bb):
        c = wid + NW * k
        return pltpu.make_async_copy(
            tab_t.at[:, pl.ds(c * LANES, LANES)], blk.at[bb], sem_i.at[bb])

    def out_desc(k, bb):
        c = wid + NW * k
        return pltpu.make_async_copy(
            pck.at[bb], out_hbm.at[pl.ds(c * D, D)], sem_o.at[bb])

    def transpose_block(bb, width):
        @plsc.parallel_loop(0, width * D // 16, unroll=8)
        def _tp(g):
            j0 = lax.div(g, 2)
            d0 = 16 * lax.rem(g, 2)
            p0 = lax.div(g, 8)
            mm0 = 16 * lax.rem(g, 8)
            vals = plsc.load_gather(
                blk.at[bb], [d0 + iota, lax.broadcast(j0, (16,))])
            pck[bb, p0, pl.ds(mm0, 16)] = vals

    in_desc(0, 0).start()

    @pl.loop(0, NK_FULL + 2, step=2)
    def _blocks(ko):
        for bb in range(2):
            k = ko + bb

            @pl.when(k < nk)
            def _step():
                @pl.when(k + 1 < nk)
                def _prefetch():
                    @pl.when(k >= 1)
                    def _():
                        out_desc(k - 1, 1 - bb).wait()
                    in_desc(k + 1, 1 - bb).start()

                in_desc(k, bb).wait()
                transpose_block(bb, LANES)
                out_desc(k, bb).start()

    for bb in range(2):
        @pl.when(lax.rem(nk, 2) == bb)
        def _drain():
            out_desc(nk - 2, bb).wait()
            out_desc(nk - 1, 1 - bb).wait()

    @pl.when(wid == NK_REM)
    def _tail():
        pltpu.sync_copy(tail4, blk.at[0, pl.ds(0, 16), :])
        pltpu.sync_copy(blk.at[0, pl.ds(0, 16), :],
                        out_hbm.at[pl.ds(NBLK * D, HALF * D // LANES)])


@functools.partial(
    pl.kernel,
    mesh=_mesh,
    compiler_params=pltpu.CompilerParams(
        use_tc_tiling_on_sc=False, needs_layout_passes=False),
    out_type=jax.ShapeDtypeStruct((T, 4, NW, 8 * LANES), jnp.float32),
    scratch_types=[
        pltpu.VMEM((LANES * T,), jnp.int32),
        pltpu.VMEM((2, LANES), jnp.int32),
        pltpu.VMEM((2, LANES, D), jnp.float32),
        pltpu.VMEM((2, 4, 8 * LANES), jnp.float32),
        pltpu.SemaphoreType.DMA((2,)),
        pltpu.SemaphoreType.DMA((2,)),
    ],
)
def _gather(idx_hbm, tab_hbm, out_hbm, idx_v, idxc, rows, otile, sem_g, sem_o):
    wid = lax.axis_index("s") * 2 + lax.axis_index("c")
    iota = lax.iota(jnp.int32, 16)
    pltpu.sync_copy(idx_hbm.at[pl.ds(wid * (LANES * T), LANES * T)], idx_v)

    def build_idxc(t, bb):
        for g in range(LANES // 16):
            base = T * 16 * g + T * iota
            idxc[bb, pl.ds(16 * g, 16)] = plsc.load_gather(idx_v, [base + t])

    def gather_desc(bb):
        return pltpu.make_async_copy(
            tab_hbm.at[idxc.at[bb]], rows.at[bb], sem_g.at[bb])

    def out_desc(t, bb):
        return pltpu.make_async_copy(
            otile.at[bb], out_hbm.at[t, :, wid], sem_o.at[bb])

    def transpose_rows(bb):
        @plsc.parallel_loop(0, D * LANES // 16, unroll=8)
        def _tp(g2):
            d = lax.div(g2, 8)
            gl = lax.rem(g2, 8)
            s = lax.div(d, 8)
            r = lax.rem(d, 8)
            vals = plsc.load_gather(
                rows.at[bb], [16 * gl + iota, lax.broadcast(d, (16,))])
            otile[bb, s, pl.ds(r * LANES + 16 * gl, 16)] = vals

    build_idxc(0, 0)
    gather_desc(0).start()

    @pl.loop(0, T, step=2)
    def _steps(to):
        for bb in range(2):
            t = to + bb

            @pl.when(t + 1 < T)
            def _prefetch():
                build_idxc(t + 1, 1 - bb)
                @pl.when(t >= 1)
                def _():
                    out_desc(t - 1, 1 - bb).wait()
                gather_desc(1 - bb).start()

            gather_desc(bb).wait()
            transpose_rows(bb)
            out_desc(t, bb).start()

    out_desc(T - 2, 0).wait()
    out_desc(T - 1, 1).wait()


def kernel(x, table):
    idx = x.reshape(-1)
    tail4 = table[NBLK * LANES:].reshape(HALF * D // LANES, LANES)
    tab_lin = _linearize(table.T, tail4).reshape(V, D)
    out4 = _gather(idx, tab_lin)
    out5 = out4.reshape(T, 4, NW, 8, LANES)
    return out5.transpose(2, 4, 0, 1, 3).reshape(B, T, D)


# speedup vs baseline: 1.5850x; 1.0465x over previous
import functools

import jax
import jax.numpy as jnp
from jax import lax
from jax.experimental import pallas as pl
from jax.experimental.pallas import tpu as pltpu
from jax.experimental.pallas import tpu_sc as plsc

D = 32
V = 1000000
B = 4096
T = 200
NW = 32
LANES = 128
NBLK = V // LANES
HALF = V - NBLK * LANES
NK_FULL = NBLK // NW
NK_REM = NBLK % NW

_mesh = plsc.VectorSubcoreMesh(core_axis_name="c", subcore_axis_name="s")


@functools.partial(
    pl.kernel,
    mesh=_mesh,
    compiler_params=pltpu.CompilerParams(needs_layout_passes=False),
    out_type=jax.ShapeDtypeStruct((V // 4, LANES), jnp.float32),
    scratch_types=[
        pltpu.VMEM((2, D, LANES), jnp.float32),
        pltpu.VMEM((2, D, LANES), jnp.float32),
        pltpu.SemaphoreType.DMA((2,)),
        pltpu.SemaphoreType.DMA((2,)),
    ],
)
def _linearize(tab_t, tail4, out_hbm, blk, pck, sem_i, sem_o):
    wid = lax.axis_index("s") * 2 + lax.axis_index("c")
    iota = lax.iota(jnp.int32, 16)
    nk = jnp.where(wid < NK_REM, NK_FULL + 1, NK_FULL)

    def in_desc(k, bb):
        c = wid + NW * k
        return pltpu.make_async_copy(
            tab_t.at[:, pl.ds(c * LANES, LANES)], blk.at[bb], sem_i.at[bb])

    def out_desc(k, bb):
        c = wid + NW * k
        return pltpu.make_async_copy(
            pck.at[bb], out_hbm.at[pl.ds(c * D, D)], sem_o.at[bb])

    def transpose_block(bb, width):
        @plsc.parallel_loop(0, width * D // 16, unroll=8)
        def _tp(g):
            j0 = lax.div(g, 2)
            d0 = 16 * lax.rem(g, 2)
            p0 = lax.div(g, 8)
            mm0 = 16 * lax.rem(g, 8)
            vals = plsc.load_gather(
                blk.at[bb], [d0 + iota, lax.broadcast(j0, (16,))])
            pck[bb, p0, pl.ds(mm0, 16)] = vals

    in_desc(0, 0).start()

    @pl.loop(0, NK_FULL + 2, step=2)
    def _blocks(ko):
        for bb in range(2):
            k = ko + bb

            @pl.when(k < nk)
            def _step():
                @pl.when(k + 1 < nk)
                def _prefetch():
                    @pl.when(k >= 1)
                    def _():
                        out_desc(k - 1, 1 - bb).wait()
                    in_desc(k + 1, 1 - bb).start()

                in_desc(k, bb).wait()
                transpose_block(bb, LANES)
                out_desc(k, bb).start()

    for bb in range(2):
        @pl.when(lax.rem(nk, 2) == bb)
        def _drain():
            out_desc(nk - 2, bb).wait()
            out_desc(nk - 1, 1 - bb).wait()

    @pl.when(wid == NK_REM)
    def _tail():
        pltpu.sync_copy(tail4, blk.at[0, pl.ds(0, 16), :])
        pltpu.sync_copy(blk.at[0, pl.ds(0, 16), :],
                        out_hbm.at[pl.ds(NBLK * D, HALF * D // LANES)])


@functools.partial(
    pl.kernel,
    mesh=_mesh,
    compiler_params=pltpu.CompilerParams(
        use_tc_tiling_on_sc=False, needs_layout_passes=False),
    out_type=jax.ShapeDtypeStruct((T, 4, NW, 8 * LANES), jnp.float32),
    scratch_types=[
        pltpu.VMEM((LANES * T,), jnp.int32),
        pltpu.VMEM((2, LANES), jnp.int32),
        pltpu.VMEM((2, LANES, D), jnp.float32),
        pltpu.VMEM((2, 4, 8 * LANES), jnp.float32),
        pltpu.SemaphoreType.DMA((2,)),
        pltpu.SemaphoreType.DMA((2,)),
    ],
)
def _gather(idx_hbm, tab_hbm, out_hbm, idx_v, idxc, rows, otile, sem_g, sem_o):
    wid = lax.axis_index("s") * 2 + lax.axis_index("c")
    iota = lax.iota(jnp.int32, 16)
    pltpu.sync_copy(idx_hbm.at[pl.ds(wid * (LANES * T), LANES * T)], idx_v)

    def build_idxc(t, bb):
        for g in range(LANES // 16):
            base = T * 16 * g + T * iota
            idxc[bb, pl.ds(16 * g, 16)] = 4 * plsc.load_gather(
                idx_v, [base + t])

    def gather_desc(bb):
        return pltpu.make_async_copy(
            tab_hbm.at[idxc.at[bb]], rows.at[bb], sem_g.at[bb])

    def out_desc(t, bb):
        return pltpu.make_async_copy(
            otile.at[bb], out_hbm.at[t, :, wid], sem_o.at[bb])

    def transpose_rows(bb):
        @plsc.parallel_loop(0, D * LANES // 16, unroll=8)
        def _tp(g2):
            d = lax.div(g2, 8)
            gl = lax.rem(g2, 8)
            s = lax.div(d, 8)
            r = lax.rem(d, 8)
            vals = plsc.load_gather(
                rows.at[bb], [16 * gl + iota, lax.broadcast(d, (16,))])
            otile[bb, s, pl.ds(r * LANES + 16 * gl, 16)] = vals

    build_idxc(0, 0)
    gather_desc(0).start()

    @pl.loop(0, T, step=2)
    def _steps(to):
        for bb in range(2):
            t = to + bb

            @pl.when(t + 1 < T)
            def _prefetch():
                build_idxc(t + 1, 1 - bb)
                @pl.when(t >= 1)
                def _():
                    out_desc(t - 1, 1 - bb).wait()
                gather_desc(1 - bb).start()

            gather_desc(bb).wait()
            transpose_rows(bb)
            out_desc(t, bb).start()

    out_desc(T - 2, 0).wait()
    out_desc(T - 1, 1).wait()


def kernel(x, table):
    idx = x.reshape(-1)
    tab_lin = jnp.pad(table, ((0, 0), (0, LANES - D))).reshape(4 * V, D)
    out4 = _gather(idx, tab_lin)
    out5 = out4.reshape(T, 4, NW, 8, LANES)
    return out5.transpose(2, 4, 0, 1, 3).reshape(B, T, D)


# speedup vs baseline: 1.8189x; 1.1476x over previous
import functools

import jax
import jax.numpy as jnp
from jax import lax
from jax.experimental import pallas as pl
from jax.experimental.pallas import tpu as pltpu
from jax.experimental.pallas import tpu_sc as plsc

D = 32
V = 1000000
B = 4096
T = 200
NW = 32
LANES = 128
NBLK = V // LANES
HALF = V - NBLK * LANES
NK_FULL = NBLK // NW
NK_REM = NBLK % NW

_mesh = plsc.VectorSubcoreMesh(core_axis_name="c", subcore_axis_name="s")


@functools.partial(
    pl.kernel,
    mesh=_mesh,
    compiler_params=pltpu.CompilerParams(
        use_tc_tiling_on_sc=False, needs_layout_passes=False),
    out_type=jax.ShapeDtypeStruct((T, 4, NW, 8 * LANES), jnp.float32),
    scratch_types=[
        pltpu.VMEM((LANES * T,), jnp.int32),
        pltpu.VMEM((2, LANES), jnp.int32),
        pltpu.VMEM((2, LANES, D), jnp.float32),
        pltpu.VMEM((2, 4, 8 * LANES), jnp.float32),
        pltpu.SemaphoreType.DMA((2,)),
        pltpu.SemaphoreType.DMA((2,)),
    ],
)
def _gather(idx_hbm, tab_hbm, out_hbm, idx_v, idxc, rows, otile, sem_g, sem_o):
    wid = lax.axis_index("s") * 2 + lax.axis_index("c")
    iota = lax.iota(jnp.int32, 16)
    pltpu.sync_copy(idx_hbm.at[pl.ds(wid * (LANES * T), LANES * T)], idx_v)

    def build_idxc(t, bb):
        for g in range(LANES // 16):
            base = T * 16 * g + T * iota
            idxc[bb, pl.ds(16 * g, 16)] = 4 * plsc.load_gather(
                idx_v, [base + t])

    def gather_desc(bb):
        return pltpu.make_async_copy(
            tab_hbm.at[idxc.at[bb]], rows.at[bb], sem_g.at[bb])

    def out_desc(t, bb):
        return pltpu.make_async_copy(
            otile.at[bb], out_hbm.at[t, :, wid], sem_o.at[bb])

    def transpose_rows(bb):
        @plsc.parallel_loop(0, D, unroll=4)
        def _tp(d):
            s = lax.div(d, 8)
            base = lax.rem(d, 8) * LANES
            dsplat = lax.broadcast(d, (16,))
            for gl in range(LANES // 16):
                vals = plsc.load_gather(
                    rows.at[bb], [16 * gl + iota, dsplat])
                otile[bb, s, pl.ds(base + 16 * gl, 16)] = vals

    build_idxc(0, 0)
    gather_desc(0).start()

    @pl.loop(0, T, step=2)
    def _steps(to):
        for bb in range(2):
            t = to + bb

            @pl.when(t + 1 < T)
            def _prefetch():
                build_idxc(t + 1, 1 - bb)
                @pl.when(t >= 1)
                def _():
                    out_desc(t - 1, 1 - bb).wait()
                gather_desc(1 - bb).start()

            gather_desc(bb).wait()
            transpose_rows(bb)
            out_desc(t, bb).start()

    out_desc(T - 2, 0).wait()
    out_desc(T - 1, 1).wait()


def kernel(x, table):
    idx = x.reshape(-1)
    tab_lin = jnp.pad(table, ((0, 0), (0, LANES - D))).reshape(4 * V, D)
    out4 = _gather(idx, tab_lin)
    out5 = out4.reshape(T, 4, NW, 8, LANES)
    return out5.transpose(2, 4, 0, 1, 3).reshape(B, T, D)
